# baseline (speedup 1.0000x reference)
import functools
import numpy as np
import jax
import jax.numpy as jnp
from jax.experimental import pallas as pl
from jax.experimental.pallas import tpu as pltpu
from jax.experimental.pallas import tpu_sc as plsc

_D = 384
_K = 8
_SIGMA = _D ** 0.5
_EPS = 1e-6


def _filter_matrix():
    pos = np.arange(-_D // 2 + 1, _D // 2 + 1, dtype=np.float64)
    g = np.exp(-0.5 * (pos / _SIGMA) ** 2)
    g = g / g.max()
    w = np.fft.ifftshift(g)
    eye = np.eye(_D, dtype=np.float64)
    m = np.fft.ifft(np.fft.fft(eye, axis=-1) * w, axis=-1).real
    return jnp.asarray(m - eye, dtype=jnp.float32)


def _body(x_ref, xp_ref, m_ref, pooled_ref, sel_ref, sbuf):
    i = pl.program_id(0)
    n = x_ref.shape[1]
    d = x_ref.shape[2]
    slot = jax.lax.rem(i, 2)
    pslot = jax.lax.rem(i + 1, 2)

    scores = sbuf[pslot]
    xp = xp_ref[0]

    x = x_ref[0]
    resid = jax.lax.dot(
        x, m_ref[...],
        precision=jax.lax.Precision.HIGHEST,
        preferred_element_type=jnp.float32,
    )
    sbuf[slot] = x / jnp.maximum(jnp.abs(resid), _EPS)

    iota = jax.lax.broadcasted_iota(jnp.int32, (n, d), 0)
    neg = jnp.float32(-jnp.inf)
    idx = None
    half = n // 2
    for k in range(_K):
        if k > 0:
            scores = jnp.where(iota == idx, neg, scores)
        s = scores
        h = half
        while h >= 8:
            s = jnp.maximum(s[:h], s[h:])
            h //= 2
        m = jnp.max(s, axis=0, keepdims=True)
        c = jnp.minimum(
            jnp.where(scores[:half] == m, iota[:half], n),
            jnp.where(scores[half:] == m, iota[half:], n),
        )
        h = half // 2
        while h >= 8:
            c = jnp.minimum(c[:h], c[h:])
            h //= 2
        idx = jnp.min(c, axis=0, keepdims=True)
        sel_ref[0, k, :] = idx[0]
    chosen = (scores == neg) | (iota == idx)
    pooled_ref[0, 0, :] = jnp.sum(jnp.where(chosen, xp, 0.0), axis=0) * (1.0 / _K)


def _make_sc_votes(b, n, kd):
    info = plsc.get_sparse_core_info()
    nw = info.num_cores * info.num_subcores
    per_w = b // nw
    lanes = info.num_lanes

    @functools.partial(
        pl.kernel,
        mesh=plsc.VectorSubcoreMesh(core_axis_name="c", subcore_axis_name="s"),
        out_type=jax.ShapeDtypeStruct((b, n), jnp.int32),
        scratch_types=[
            pltpu.VMEM((kd,), jnp.int32),
            pltpu.VMEM((n,), jnp.int32),
        ],
        compiler_params=pltpu.CompilerParams(needs_layout_passes=False),
    )
    def sc_votes(sel_hbm, out_hbm, idx_v, votes_v):
        wid = jax.lax.axis_index("s") * info.num_cores + jax.lax.axis_index("c")
        ones = jnp.ones((lanes,), jnp.int32)
        zeros = jnp.zeros((lanes,), jnp.int32)
        for t in range(per_w):
            row = wid * per_w + t
            pltpu.sync_copy(sel_hbm.at[row], idx_v)

            def zero_body(j, carry):
                votes_v[pl.ds(j * lanes, lanes)] = zeros
                return carry

            jax.lax.fori_loop(0, n // lanes, zero_body, 0)

            def scat_body(j, carry):
                iv = idx_v[pl.ds(j * lanes, lanes)]
                plsc.addupdate_scatter(votes_v, [iv], ones)
                return carry

            jax.lax.fori_loop(0, kd // lanes, scat_body, 0)
            pltpu.sync_copy(votes_v, out_hbm.at[row])

    return sc_votes


def kernel(patch_tokens):
    b, n, d = patch_tokens.shape
    m = _filter_matrix()
    pooled, sel = pl.pallas_call(
        _body,
        grid=(b + 1,),
        in_specs=[
            pl.BlockSpec((1, n, d), lambda i: (jnp.minimum(i, b - 1), 0, 0)),
            pl.BlockSpec((1, n, d), lambda i: (jnp.maximum(i - 1, 0), 0, 0)),
            pl.BlockSpec((d, d), lambda i: (0, 0)),
        ],
        out_specs=[
            pl.BlockSpec((1, 1, d), lambda i: (jnp.maximum(i - 1, 0), 0, 0)),
            pl.BlockSpec((1, _K, d), lambda i: (jnp.maximum(i - 1, 0), 0, 0)),
        ],
        out_shape=[
            jax.ShapeDtypeStruct((b, 1, d), jnp.float32),
            jax.ShapeDtypeStruct((b, _K, d), jnp.int32),
        ],
        scratch_shapes=[
            pltpu.VMEM((2, n, d), jnp.float32),
        ],
        compiler_params=pltpu.CompilerParams(
            dimension_semantics=("arbitrary",),
        ),
    )(patch_tokens, patch_tokens, m)
    votes = _make_sc_votes(b, n, _K * d)(sel.reshape(b, _K * d))
    return pooled.reshape(b, d), votes, sel
